# baseline (speedup 1.0000x reference)
import functools

import jax
import jax.numpy as jnp
from jax import lax
from jax.experimental import pallas as pl
from jax.experimental.pallas import tpu as pltpu
from jax.experimental.pallas import tpu_sc as plsc

B, T, H, D = 1024, 20, 26, 32
NC, NS = 2, 16
NW = NC * NS
RPW = B // NW
TP, HP = 24, 128


def _tc_shift_body(ids_ref, offs_ref, out_ref):
    out_ref[:, :T, :H] = ids_ref[...] + offs_ref[...]
    out_ref[:, :T, H:D] = jnp.zeros((B, T, D - H), jnp.int32)


_tc_shift = pl.pallas_call(
    _tc_shift_body,
    out_shape=jax.ShapeDtypeStruct((B, TP, HP), jnp.int32),
)


def _sc_body(ids_hbm, table_hbm, out_hbm, idx0_v, idx1_v, rows0_v, rows1_v,
             sem_g, sem_o):
    wid = lax.axis_index("s") * NC + lax.axis_index("c")

    def stage(b, idx_v):
        pltpu.sync_copy(ids_hbm.at[b, pl.ds(0, T), pl.ds(0, 32)], idx_v)

    def fire(idx_v, buf):
        for t in range(T):
            pltpu.async_copy(table_hbm.at[idx_v.at[t]], buf.at[t], sem_g)

    def drain(idx_v, buf):
        for t in range(T):
            pltpu.make_async_copy(table_hbm.at[idx_v.at[t]], buf.at[t],
                                  sem_g).wait()

    def out_copy(b, buf):
        pltpu.async_copy(buf.at[:, pl.ds(0, H), :], out_hbm.at[b], sem_o)

    def out_wait(b, buf):
        pltpu.make_async_copy(buf.at[:, pl.ds(0, H), :], out_hbm.at[b],
                              sem_o).wait()

    b0 = wid * RPW
    stage(b0, idx0_v)
    fire(idx0_v, rows0_v)

    def pair(p, carry):
        b = b0 + p * 2
        stage(b + 1, idx1_v)
        fire(idx1_v, rows1_v)
        drain(idx0_v, rows0_v)
        out_copy(b, rows0_v)

        @pl.when(p + 1 < RPW // 2)
        def _():
            stage(b + 2, idx0_v)
            out_wait(b, rows0_v)
            fire(idx0_v, rows0_v)

        drain(idx1_v, rows1_v)
        out_copy(b + 1, rows1_v)

        @pl.when(p + 1 < RPW // 2)
        def _():
            out_wait(b + 1, rows1_v)

        return carry

    lax.fori_loop(0, RPW // 2, pair, 0)
    out_wait(b0 + RPW - 2, rows0_v)
    out_wait(b0 + RPW - 1, rows1_v)


@functools.partial(
    pl.kernel,
    out_type=jax.ShapeDtypeStruct((B, T, H, D), jnp.float32),
    mesh=plsc.VectorSubcoreMesh(core_axis_name="c", subcore_axis_name="s"),
    scratch_types=[
        pltpu.VMEM((T, 32), jnp.int32),
        pltpu.VMEM((T, 32), jnp.int32),
        pltpu.VMEM((T, 32, D), jnp.float32),
        pltpu.VMEM((T, 32, D), jnp.float32),
        pltpu.SemaphoreType.DMA,
        pltpu.SemaphoreType.DMA,
    ],
    compiler_params=pltpu.CompilerParams(use_tc_tiling_on_sc=False),
)
def _sc_gather(ids_hbm, table_hbm, out_hbm, idx0_v, idx1_v, rows0_v,
               rows1_v, sem_g, sem_o):
    _sc_body(ids_hbm, table_hbm, out_hbm, idx0_v, idx1_v, rows0_v, rows1_v,
             sem_g, sem_o)


def kernel(input_ids, table, offsets):
    shifted = _tc_shift(input_ids, offsets.reshape(1, 1, H))
    return _sc_gather(shifted, table)
